# Initial kernel scaffold; baseline (speedup 1.0000x reference)
import jax
import jax.numpy as jnp
from jax.experimental import pallas as pl


def kernel(node_feat, batch):
    raise NotImplementedError("write your pallas kernel here")


# speedup vs baseline: 2.1053x; 2.1053x over previous
import functools

import jax
import jax.numpy as jnp
from jax import lax
from jax.experimental import pallas as pl
from jax.experimental.pallas import tpu as pltpu
from jax.experimental.pallas import tpu_sc as plsc

N_NODES = 100000
D_FEAT = 128
NUM_GRAPHS = 512
NUM_WORKERS = 32
SEGS_PER_W = NUM_GRAPHS // NUM_WORKERS
BLK = 256
LANES = 16
NBOUNDS = 48


def _make_kernel():
    mesh = plsc.VectorSubcoreMesh(core_axis_name="c", subcore_axis_name="s")

    @functools.partial(
        pl.kernel,
        mesh=mesh,
        out_type=jax.ShapeDtypeStruct((NUM_GRAPHS, D_FEAT), jnp.float32),
        scratch_types=[
            pltpu.VMEM((NBOUNDS,), jnp.int32),
            pltpu.VMEM((BLK,), jnp.int32),
            pltpu.VMEM((BLK, D_FEAT), jnp.float32),
            pltpu.VMEM((SEGS_PER_W, D_FEAT), jnp.float32),
        ],
    )
    def seg_sum(feat_hbm, batch_hbm, bounds_hbm, out_hbm,
                bounds_v, ids_v, rows_v, acc_v):
        wid = lax.axis_index("s") * 2 + lax.axis_index("c")
        seg_base = wid * SEGS_PER_W

        pltpu.sync_copy(bounds_hbm, bounds_v)

        zeros = jnp.zeros((LANES,), jnp.float32)
        for s in range(SEGS_PER_W):
            for j in range(D_FEAT // LANES):
                acc_v[s, pl.ds(j * LANES, LANES)] = zeros

        def pick(idx):
            total = jnp.zeros((), jnp.int32)
            for c in range(NBOUNDS // LANES):
                if c * LANES >= NUM_WORKERS + 1:
                    break
                bv = bounds_v[pl.ds(c * LANES, LANES)]
                for k in range(LANES):
                    pos = c * LANES + k
                    if pos >= NUM_WORKERS + 2:
                        break
                    total = jnp.where(idx == pos, bv[k], total)
            return total

        r_lo = pick(wid)
        r_hi = pick(wid + 1)
        a_lo = (r_lo // 8) * 8
        nblk = (r_hi - a_lo + BLK - 1) // BLK

        def blk_body(b, _):
            unclamped = a_lo + b * BLK
            start = jnp.minimum(unclamped, N_NODES - BLK)
            pltpu.sync_copy(batch_hbm.at[pl.ds(start, BLK)], ids_v)
            pltpu.sync_copy(feat_hbm.at[pl.ds(start, BLK)], rows_v)
            lo = jnp.maximum(jnp.maximum(r_lo, unclamped) - start, 0)
            hi = jnp.minimum(r_hi - start, BLK)

            def grp_body(g, _):
                base_i = g * LANES
                idv = ids_v[pl.ds(base_i, LANES)] - seg_base
                for k in range(LANES):
                    i = base_i + k
                    sid = idv[k]
                    ok = jnp.logical_and(i >= lo, i < hi)

                    @pl.when(ok)
                    def _():
                        for j in range(D_FEAT // LANES):
                            plsc.addupdate(
                                acc_v.at[sid, pl.ds(j * LANES, LANES)],
                                rows_v[i, pl.ds(j * LANES, LANES)],
                            )
                return 0

            lax.fori_loop(0, BLK // LANES, grp_body, 0, unroll=False)
            return 0

        lax.fori_loop(0, nblk, blk_body, 0, unroll=False)

        pltpu.sync_copy(acc_v, out_hbm.at[pl.ds(seg_base, SEGS_PER_W)])

    return seg_sum


_SEG_SUM = _make_kernel()


@jax.jit
def kernel(node_feat, batch):
    batch = batch.astype(jnp.int32)
    qs = jnp.arange(0, NUM_GRAPHS + 1, SEGS_PER_W, dtype=jnp.int32)
    bounds = jnp.searchsorted(batch, qs).astype(jnp.int32)
    bounds = jnp.concatenate(
        [bounds, jnp.zeros((NBOUNDS - NUM_WORKERS - 1,), jnp.int32)])
    return _SEG_SUM(node_feat, batch, bounds)


# speedup vs baseline: 3.7306x; 1.7720x over previous
import functools

import jax
import jax.numpy as jnp
from jax import lax
from jax.experimental import pallas as pl
from jax.experimental.pallas import tpu as pltpu
from jax.experimental.pallas import tpu_sc as plsc

N_NODES = 100000
D_FEAT = 128
NUM_GRAPHS = 512
NUM_WORKERS = 32
SEGS_PER_W = NUM_GRAPHS // NUM_WORKERS
BLK = 256
LANES = 16
NBOUNDS = 48


def _make_kernel():
    mesh = plsc.VectorSubcoreMesh(core_axis_name="c", subcore_axis_name="s")

    @functools.partial(
        pl.kernel,
        mesh=mesh,
        out_type=jax.ShapeDtypeStruct((NUM_GRAPHS, D_FEAT), jnp.float32),
        scratch_types=[
            pltpu.VMEM((NBOUNDS,), jnp.int32),
            pltpu.VMEM((BLK,), jnp.int32),
            pltpu.VMEM((BLK,), jnp.int32),
            pltpu.VMEM((BLK, D_FEAT), jnp.float32),
            pltpu.VMEM((BLK, D_FEAT), jnp.float32),
            pltpu.VMEM((SEGS_PER_W, D_FEAT), jnp.float32),
            pltpu.SemaphoreType.DMA,
            pltpu.SemaphoreType.DMA,
            pltpu.SemaphoreType.DMA,
            pltpu.SemaphoreType.DMA,
        ],
    )
    def seg_sum(feat_hbm, batch_hbm, bounds_hbm, out_hbm,
                bounds_v, ids_a, ids_b, rows_a, rows_b, acc_v,
                si0, sr0, si1, sr1):
        wid = lax.axis_index("s") * 2 + lax.axis_index("c")
        seg_base = wid * SEGS_PER_W

        pltpu.sync_copy(bounds_hbm, bounds_v)

        zeros = jnp.zeros((LANES,), jnp.float32)
        for s in range(SEGS_PER_W):
            for j in range(D_FEAT // LANES):
                acc_v[s, pl.ds(j * LANES, LANES)] = zeros

        def pick(idx):
            total = jnp.zeros((), jnp.int32)
            for c in range(NBOUNDS // LANES):
                if c * LANES >= NUM_WORKERS + 1:
                    break
                bv = bounds_v[pl.ds(c * LANES, LANES)]
                for k in range(LANES):
                    pos = c * LANES + k
                    if pos >= NUM_WORKERS + 2:
                        break
                    total = jnp.where(idx == pos, bv[k], total)
            return total

        r_lo = pick(wid)
        r_hi = pick(wid + 1)
        a_lo = (r_lo // 8) * 8
        nblk = (r_hi - a_lo + BLK - 1) // BLK

        def start_of(b):
            unclamped = a_lo + b * BLK
            return unclamped, jnp.minimum(unclamped, N_NODES - BLK)

        def issue(b, idsbuf, rowsbuf, semi, semr):
            _, start = start_of(b)
            pltpu.make_async_copy(
                batch_hbm.at[pl.ds(start, BLK)], idsbuf, semi).start()
            pltpu.make_async_copy(
                feat_hbm.at[pl.ds(start, BLK)], rowsbuf, semr).start()

        def wait(idsbuf, rowsbuf, semi, semr):
            pltpu.make_async_copy(
                batch_hbm.at[pl.ds(0, BLK)], idsbuf, semi).wait()
            pltpu.make_async_copy(
                feat_hbm.at[pl.ds(0, BLK)], rowsbuf, semr).wait()

        def accum_row(idsbuf, rowsbuf, i, sid):
            for j in range(D_FEAT // LANES):
                plsc.addupdate(
                    acc_v.at[sid, pl.ds(j * LANES, LANES)],
                    rowsbuf[i, pl.ds(j * LANES, LANES)],
                )

        def process(b, idsbuf, rowsbuf):
            unclamped, start = start_of(b)
            lo = jnp.maximum(jnp.maximum(r_lo, unclamped) - start, 0)
            hi = jnp.minimum(r_hi - start, BLK)
            full = jnp.logical_and(lo == 0, hi == BLK)

            def fast_grp(g, _):
                base_i = g * LANES
                idv = idsbuf[pl.ds(base_i, LANES)] - seg_base
                uniform = idv[0] == idv[LANES - 1]

                def uni():
                    sid = idv[0]
                    for j in range(D_FEAT // LANES):
                        sl = pl.ds(j * LANES, LANES)
                        vs = [rowsbuf[base_i + k, sl] for k in range(LANES)]
                        while len(vs) > 1:
                            nxt = [vs[m] + vs[m + 1]
                                   for m in range(0, len(vs) - 1, 2)]
                            if len(vs) % 2:
                                nxt.append(vs[-1])
                            vs = nxt
                        plsc.addupdate(acc_v.at[sid, sl], vs[0])
                    return 0

                def mixed():
                    for k in range(LANES):
                        accum_row(idsbuf, rowsbuf, base_i + k, idv[k])
                    return 0

                lax.cond(uniform, uni, mixed)
                return 0

            def guard_grp(g, _):
                base_i = g * LANES
                idv = idsbuf[pl.ds(base_i, LANES)] - seg_base
                for k in range(LANES):
                    i = base_i + k
                    ok = jnp.logical_and(i >= lo, i < hi)

                    @pl.when(ok)
                    def _():
                        accum_row(idsbuf, rowsbuf, i, idv[k])
                return 0

            lax.cond(
                full,
                lambda: lax.fori_loop(0, BLK // LANES, fast_grp, 0,
                                      unroll=False),
                lambda: lax.fori_loop(0, BLK // LANES, guard_grp, 0,
                                      unroll=False),
            )

        @pl.when(nblk > 0)
        def _():
            issue(0, ids_a, rows_a, si0, sr0)

        npairs = (nblk + 1) // 2

        def pair(p, _):
            b0 = 2 * p
            b1 = b0 + 1

            @pl.when(b1 < nblk)
            def _():
                issue(b1, ids_b, rows_b, si1, sr1)

            wait(ids_a, rows_a, si0, sr0)
            process(b0, ids_a, rows_a)

            @pl.when(b0 + 2 < nblk)
            def _():
                issue(b0 + 2, ids_a, rows_a, si0, sr0)

            @pl.when(b1 < nblk)
            def _():
                wait(ids_b, rows_b, si1, sr1)
                process(b1, ids_b, rows_b)

            return 0

        lax.fori_loop(0, npairs, pair, 0, unroll=False)

        pltpu.sync_copy(acc_v, out_hbm.at[pl.ds(seg_base, SEGS_PER_W)])

    return seg_sum


_SEG_SUM = _make_kernel()


@jax.jit
def kernel(node_feat, batch):
    batch = batch.astype(jnp.int32)
    qs = jnp.arange(0, NUM_GRAPHS + 1, SEGS_PER_W, dtype=jnp.int32)
    bounds = jnp.searchsorted(batch, qs).astype(jnp.int32)
    bounds = jnp.concatenate(
        [bounds, jnp.zeros((NBOUNDS - NUM_WORKERS - 1,), jnp.int32)])
    return _SEG_SUM(node_feat, batch, bounds)


# speedup vs baseline: 3.7316x; 1.0003x over previous
import functools

import jax
import jax.numpy as jnp
from jax import lax
from jax.experimental import pallas as pl
from jax.experimental.pallas import tpu as pltpu
from jax.experimental.pallas import tpu_sc as plsc

N_NODES = 100000
D_FEAT = 128
NUM_GRAPHS = 512
NUM_WORKERS = 32
SEGS_PER_W = NUM_GRAPHS // NUM_WORKERS
BLK = 256
LANES = 16
NBOUNDS = 48


def _make_kernel():
    mesh = plsc.VectorSubcoreMesh(core_axis_name="c", subcore_axis_name="s")

    @functools.partial(
        pl.kernel,
        mesh=mesh,
        out_type=jax.ShapeDtypeStruct((NUM_GRAPHS, D_FEAT), jnp.float32),
        scratch_types=[
            pltpu.VMEM((NBOUNDS,), jnp.int32),
            pltpu.VMEM((BLK,), jnp.int32),
            pltpu.VMEM((BLK,), jnp.int32),
            pltpu.VMEM((BLK, D_FEAT), jnp.float32),
            pltpu.VMEM((BLK, D_FEAT), jnp.float32),
            pltpu.VMEM((SEGS_PER_W, D_FEAT), jnp.float32),
            pltpu.SemaphoreType.DMA,
            pltpu.SemaphoreType.DMA,
            pltpu.SemaphoreType.DMA,
            pltpu.SemaphoreType.DMA,
        ],
    )
    def seg_sum(feat_hbm, batch_hbm, bounds_hbm, out_hbm,
                bounds_v, ids_a, ids_b, rows_a, rows_b, acc_v,
                si0, sr0, si1, sr1):
        wid = lax.axis_index("s") * 2 + lax.axis_index("c")
        seg_base = wid * SEGS_PER_W

        pltpu.sync_copy(bounds_hbm, bounds_v)

        zeros = jnp.zeros((LANES,), jnp.float32)
        for s in range(SEGS_PER_W):
            for j in range(D_FEAT // LANES):
                acc_v[s, pl.ds(j * LANES, LANES)] = zeros

        def pick(idx):
            total = jnp.zeros((), jnp.int32)
            for c in range(NBOUNDS // LANES):
                if c * LANES >= NUM_WORKERS + 1:
                    break
                bv = bounds_v[pl.ds(c * LANES, LANES)]
                for k in range(LANES):
                    pos = c * LANES + k
                    if pos >= NUM_WORKERS + 2:
                        break
                    total = jnp.where(idx == pos, bv[k], total)
            return total

        r_lo = pick(wid)
        r_hi = pick(wid + 1)
        a_lo = (r_lo // 8) * 8
        nblk = (r_hi - a_lo + BLK - 1) // BLK

        def start_of(b):
            unclamped = a_lo + b * BLK
            return unclamped, jnp.minimum(unclamped, N_NODES - BLK)

        def issue(b, idsbuf, rowsbuf, semi, semr):
            _, start = start_of(b)
            pltpu.make_async_copy(
                batch_hbm.at[pl.ds(start, BLK)], idsbuf, semi).start()
            pltpu.make_async_copy(
                feat_hbm.at[pl.ds(start, BLK)], rowsbuf, semr).start()

        def wait(idsbuf, rowsbuf, semi, semr):
            pltpu.make_async_copy(
                batch_hbm.at[pl.ds(0, BLK)], idsbuf, semi).wait()
            pltpu.make_async_copy(
                feat_hbm.at[pl.ds(0, BLK)], rowsbuf, semr).wait()

        def accum_row(idsbuf, rowsbuf, i, sid):
            for j in range(D_FEAT // LANES):
                plsc.addupdate(
                    acc_v.at[sid, pl.ds(j * LANES, LANES)],
                    rowsbuf[i, pl.ds(j * LANES, LANES)],
                )

        def process(b, idsbuf, rowsbuf):
            unclamped, start = start_of(b)
            lo = jnp.maximum(jnp.maximum(r_lo, unclamped) - start, 0)
            hi = jnp.minimum(r_hi - start, BLK)
            full = jnp.logical_and(lo == 0, hi == BLK)

            def fast_grp(g, _):
                base_i = g * LANES
                idv = idsbuf[pl.ds(base_i, LANES)] - seg_base
                uniform = idv[0] == idv[LANES - 1]

                def uni():
                    sid = idv[0]
                    for j in range(D_FEAT // LANES):
                        sl = pl.ds(j * LANES, LANES)
                        vs = [rowsbuf[base_i + k, sl] for k in range(LANES)]
                        while len(vs) > 1:
                            nxt = [vs[m] + vs[m + 1]
                                   for m in range(0, len(vs) - 1, 2)]
                            if len(vs) % 2:
                                nxt.append(vs[-1])
                            vs = nxt
                        plsc.addupdate(acc_v.at[sid, sl], vs[0])
                    return 0

                def mixed():
                    for k in range(LANES):
                        accum_row(idsbuf, rowsbuf, base_i + k, idv[k])
                    return 0

                lax.cond(uniform, uni, mixed)
                return 0

            def guard_grp(g, _):
                base_i = g * LANES
                idv = idsbuf[pl.ds(base_i, LANES)] - seg_base
                for k in range(LANES):
                    i = base_i + k
                    ok = jnp.logical_and(i >= lo, i < hi)

                    @pl.when(ok)
                    def _():
                        accum_row(idsbuf, rowsbuf, i, idv[k])
                return 0

            lax.cond(
                full,
                lambda: lax.fori_loop(0, BLK // LANES, fast_grp, 0,
                                      unroll=False),
                lambda: lax.fori_loop(0, BLK // LANES, guard_grp, 0,
                                      unroll=False),
            )

        @pl.when(nblk > 0)
        def _():
            issue(0, ids_a, rows_a, si0, sr0)

        npairs = (nblk + 1) // 2

        def pair(p, _):
            b0 = 2 * p
            b1 = b0 + 1

            @pl.when(b1 < nblk)
            def _():
                issue(b1, ids_b, rows_b, si1, sr1)

            wait(ids_a, rows_a, si0, sr0)
            process(b0, ids_a, rows_a)

            @pl.when(b0 + 2 < nblk)
            def _():
                issue(b0 + 2, ids_a, rows_a, si0, sr0)

            @pl.when(b1 < nblk)
            def _():
                wait(ids_b, rows_b, si1, sr1)
                process(b1, ids_b, rows_b)

            return 0

        lax.fori_loop(0, npairs, pair, 0, unroll=False)

        pltpu.sync_copy(acc_v, out_hbm.at[pl.ds(seg_base, SEGS_PER_W)])

    return seg_sum


_SEG_SUM = _make_kernel()


@jax.jit
def kernel(node_feat, batch):
    batch = batch.astype(jnp.int32)
    qs = jnp.arange(0, NUM_GRAPHS + 1, SEGS_PER_W, dtype=jnp.int32)
    bounds = jnp.searchsorted(batch, qs, side="left").astype(jnp.int32)
    bounds = jnp.concatenate(
        [bounds, jnp.zeros((NBOUNDS - NUM_WORKERS - 1,), jnp.int32)])
    return _SEG_SUM(node_feat, batch, bounds)


# speedup vs baseline: 4.5740x; 1.2258x over previous
import functools

import jax
import jax.numpy as jnp
from jax import lax
from jax.experimental import pallas as pl
from jax.experimental.pallas import tpu as pltpu
from jax.experimental.pallas import tpu_sc as plsc

N_NODES = 100000
D_FEAT = 128
NUM_GRAPHS = 512
NUM_WORKERS = 32
SEGS_PER_W = NUM_GRAPHS // NUM_WORKERS
BLK = 128
LANES = 16
NBOUNDS = 48
NBUF = 4
NSLOT = 8
SLOT_STRIDE = NUM_GRAPHS + 1


def _make_kernel():
    mesh = plsc.VectorSubcoreMesh(core_axis_name="c", subcore_axis_name="s")

    @functools.partial(
        pl.kernel,
        mesh=mesh,
        out_type=jax.ShapeDtypeStruct((NUM_GRAPHS, D_FEAT), jnp.float32),
        scratch_types=[
            pltpu.VMEM((NBOUNDS,), jnp.int32),
            pltpu.VMEM((BLK,), jnp.int32),
            pltpu.VMEM((BLK,), jnp.int32),
            pltpu.VMEM((BLK,), jnp.int32),
            pltpu.VMEM((BLK,), jnp.int32),
            pltpu.VMEM((BLK, D_FEAT), jnp.float32),
            pltpu.VMEM((BLK, D_FEAT), jnp.float32),
            pltpu.VMEM((BLK, D_FEAT), jnp.float32),
            pltpu.VMEM((BLK, D_FEAT), jnp.float32),
            pltpu.VMEM((BLK,), jnp.int32),
            pltpu.VMEM((BLK,), jnp.int32),
            pltpu.VMEM((BLK,), jnp.int32),
            pltpu.VMEM((BLK,), jnp.int32),
            pltpu.VMEM((SEGS_PER_W, D_FEAT), jnp.float32),
            pltpu.VMEM((SEGS_PER_W, D_FEAT), jnp.float32),
            pltpu.VMEM_SHARED((NSLOT * SLOT_STRIDE, D_FEAT), jnp.float32),
            pltpu.SemaphoreType.DMA,
            pltpu.SemaphoreType.DMA,
            pltpu.SemaphoreType.DMA,
            pltpu.SemaphoreType.DMA,
            pltpu.SemaphoreType.DMA,
            pltpu.SemaphoreType.DMA,
            pltpu.SemaphoreType.DMA,
            pltpu.SemaphoreType.DMA,
            pltpu.SemaphoreType.DMA,
            pltpu.SemaphoreType.DMA,
            pltpu.SemaphoreType.DMA,
            pltpu.SemaphoreType.DMA,
        ],
    )
    def seg_sum(feat_hbm, batch_hbm, bounds_hbm, out_hbm,
                bounds_v, ids_a, ids_b, ids_c, ids_d,
                rows_a, rows_b, rows_c, rows_d,
                idx_a, idx_b, idx_c, idx_d, zeros_v, mtmp, acc_sh,
                si0, si1, si2, si3, sr0, sr1, sr2, sr3,
                ss0, ss1, ss2, ss3):
        wid = lax.axis_index("s") * 2 + lax.axis_index("c")
        seg_base = wid * SEGS_PER_W

        pltpu.sync_copy(bounds_hbm, bounds_v)

        zeros = jnp.zeros((LANES,), jnp.float32)
        for s in range(SEGS_PER_W):
            for j in range(D_FEAT // LANES):
                zeros_v[s, pl.ds(j * LANES, LANES)] = zeros
        for s in range(NSLOT):
            pltpu.sync_copy(
                zeros_v,
                acc_sh.at[pl.ds(1 + seg_base + s * SLOT_STRIDE,
                                SEGS_PER_W)])

        def pick(idx):
            total = jnp.zeros((), jnp.int32)
            for c in range(NBOUNDS // LANES):
                if c * LANES >= NUM_WORKERS + 1:
                    break
                bv = bounds_v[pl.ds(c * LANES, LANES)]
                for k in range(LANES):
                    pos = c * LANES + k
                    if pos >= NUM_WORKERS + 2:
                        break
                    total = jnp.where(idx == pos, bv[k], total)
            return total

        r_lo = pick(wid)
        r_hi = pick(wid + 1)
        a_lo = (r_lo // 8) * 8
        nblk = (r_hi - a_lo + BLK - 1) // BLK

        def start_of(b):
            unclamped = a_lo + b * BLK
            return unclamped, jnp.minimum(unclamped, N_NODES - BLK)

        bufs = [(ids_a, rows_a, idx_a, si0, sr0, ss0),
                (ids_b, rows_b, idx_b, si1, sr1, ss1),
                (ids_c, rows_c, idx_c, si2, sr2, ss2),
                (ids_d, rows_d, idx_d, si3, sr3, ss3)]

        def issue_load(b, buf):
            ids, rows, _, semi, semr, _ = buf
            _, start = start_of(b)
            pltpu.make_async_copy(
                batch_hbm.at[pl.ds(start, BLK)], ids, semi).start()
            pltpu.make_async_copy(
                feat_hbm.at[pl.ds(start, BLK)], rows, semr).start()

        def wait_load(buf):
            ids, rows, _, semi, semr, _ = buf
            pltpu.make_async_copy(
                batch_hbm.at[pl.ds(0, BLK)], ids, semi).wait()
            pltpu.make_async_copy(
                feat_hbm.at[pl.ds(0, BLK)], rows, semr).wait()

        def build_idx(b, buf):
            ids, _, idx, _, _, _ = buf
            unclamped, start = start_of(b)
            lo = jnp.maximum(jnp.maximum(r_lo, unclamped) - start, 0)
            hi = jnp.minimum(r_hi - start, BLK)
            slot_off = (jnp.arange(LANES, dtype=jnp.int32) % NSLOT
                        ) * SLOT_STRIDE
            for c in range(BLK // LANES):
                idv = ids[pl.ds(c * LANES, LANES)] + 1
                pos = jnp.arange(c * LANES, (c + 1) * LANES,
                                 dtype=jnp.int32)
                ok = jnp.logical_and(pos >= lo, pos < hi)
                idx[pl.ds(c * LANES, LANES)] = (
                    jnp.where(ok, idv, 0) + slot_off)

        def issue_scatter(buf):
            _, rows, idx, _, _, sems = buf
            pltpu.make_async_copy(rows, acc_sh.at[idx], sems).start(add=True)

        def wait_scatter(buf):
            _, rows, idx, _, _, sems = buf
            pltpu.make_async_copy(rows, acc_sh.at[idx], sems).wait()

        for p in range(NBUF - 1):
            @pl.when(nblk > p)
            def _(p=p):
                issue_load(p, bufs[p])

        def step(b, X, W):
            @pl.when(b < nblk)
            def _():
                wait_load(X)
                build_idx(b, X)

                @pl.when(b >= 1)
                def _():
                    wait_scatter(W)

                issue_scatter(X)

                @pl.when(b + NBUF - 1 < nblk)
                def _():
                    issue_load(b + NBUF - 1, W)

        ntrip = (nblk + NBUF - 1) // NBUF

        def trip(t, _):
            for k in range(NBUF):
                step(NBUF * t + k, bufs[k], bufs[(k + NBUF - 1) % NBUF])
            return 0

        lax.fori_loop(0, ntrip, trip, 0, unroll=False)

        def drain(k):
            @pl.when(jnp.logical_and(nblk > 0, (nblk - 1) % NBUF == k))
            def _():
                wait_scatter(bufs[k])

        for k in range(NBUF):
            drain(k)

        pltpu.sync_copy(
            acc_sh.at[pl.ds(1 + seg_base, SEGS_PER_W)], zeros_v)
        for s in range(1, NSLOT):
            pltpu.sync_copy(
                acc_sh.at[pl.ds(1 + seg_base + s * SLOT_STRIDE,
                                SEGS_PER_W)], mtmp)
            for r in range(SEGS_PER_W):
                for j in range(D_FEAT // LANES):
                    sl = pl.ds(j * LANES, LANES)
                    zeros_v[r, sl] = zeros_v[r, sl] + mtmp[r, sl]

        pltpu.sync_copy(
            zeros_v, out_hbm.at[pl.ds(seg_base, SEGS_PER_W)])

    return seg_sum


_SEG_SUM = _make_kernel()


@jax.jit
def kernel(node_feat, batch):
    batch = batch.astype(jnp.int32)
    qs = jnp.arange(0, NUM_GRAPHS + 1, SEGS_PER_W, dtype=jnp.int32)
    bounds = jnp.searchsorted(batch, qs, side="left").astype(jnp.int32)
    bounds = jnp.concatenate(
        [bounds, jnp.zeros((NBOUNDS - NUM_WORKERS - 1,), jnp.int32)])
    return _SEG_SUM(node_feat, batch, bounds)
